# baseline (speedup 1.0000x reference)
import functools

import jax
import jax.numpy as jnp
from jax import lax
from jax.experimental import pallas as pl
from jax.experimental.pallas import tpu as pltpu
from jax.experimental.pallas import tpu_sc as plsc

NC = 2
NS = 16
NW = NC * NS
LANES = 16

D = 64
DC = D // LANES
K = 20
CHUNK = 16
SBB = 4
N_SB = CHUNK // SBB
C_ROWS = CHUNK * K
C_STREAM = SBB * K


def _make_sc_call(B):
    b_per_w = B // NW
    n_chunks = b_per_w // CHUNK
    mesh = plsc.VectorSubcoreMesh(core_axis_name="c", subcore_axis_name="s")

    @functools.partial(
        pl.kernel,
        out_type=jax.ShapeDtypeStruct((B * K,), jnp.float32),
        mesh=mesh,
        scratch_types=[
            pltpu.VMEM((CHUNK,), jnp.int32),
            pltpu.VMEM((C_ROWS,), jnp.int32),
            pltpu.VMEM((CHUNK, D), jnp.float32),
            pltpu.VMEM((C_ROWS, D), jnp.float32),
            pltpu.VMEM((C_ROWS,), jnp.float32),
            pltpu.SemaphoreType.DMA,
        ],
        compiler_params=pltpu.CompilerParams(needs_layout_passes=False,
                                             use_tc_tiling_on_sc=False),
    )
    def sc_kernel(tw_hbm, cw_hbm, ttab_hbm, ctab_hbm, out_hbm,
                  tidx_v, cidx_v, trows_v, crows_v, outv, sem):
        wid = lax.axis_index("s") * NC + lax.axis_index("c")
        lane = lax.iota(jnp.int32, LANES)

        def chunk_body(ci, _):
            b0 = wid * b_per_w + ci * CHUNK
            pltpu.sync_copy(tw_hbm.at[pl.ds(b0, CHUNK)], tidx_v)
            pltpu.sync_copy(cw_hbm.at[pl.ds(b0 * K, C_ROWS)], cidx_v)

            cps = [pltpu.async_copy(ttab_hbm.at[tidx_v], trows_v, sem)]
            for s in range(C_ROWS // C_STREAM):
                cps.append(pltpu.async_copy(
                    ctab_hbm.at[cidx_v.at[pl.ds(s * C_STREAM, C_STREAM)]],
                    crows_v.at[pl.ds(s * C_STREAM, C_STREAM)],
                    sem))
            for cp in cps:
                cp.wait()

            def sb_body(sb, _):
                tv = {}
                for bb in range(SBB):
                    for dc in range(DC):
                        tv[(bb, dc)] = trows_v[sb * SBB + bb,
                                               pl.ds(dc * LANES, LANES)]
                for g in range(SBB * K // LANES):
                    outvec = jnp.zeros((LANES,), jnp.float32)
                    for j in range(LANES):
                        flat = g * LANES + j
                        bb, k = divmod(flat, K)
                        row = sb * C_STREAM + flat
                        acc = tv[(bb, 0)] * crows_v[row, pl.ds(0, LANES)]
                        for dc in range(1, DC):
                            acc = acc + tv[(bb, dc)] * crows_v[
                                row, pl.ds(dc * LANES, LANES)]
                        outvec = jnp.where(lane == j, jnp.sum(acc), outvec)
                    outv[pl.ds(sb * C_STREAM + g * LANES, LANES)] = outvec
                return 0

            lax.fori_loop(0, N_SB, sb_body, 0, unroll=False)
            pltpu.sync_copy(outv, out_hbm.at[pl.ds(b0 * K, C_ROWS)])
            return 0

        lax.fori_loop(0, n_chunks, chunk_body, 0, unroll=False)

    return sc_kernel


def kernel(target_word, context_word, target_table, context_table):
    B, k = context_word.shape
    assert k == K and target_table.shape[1] == D
    sc_call = _make_sc_call(B)
    out = sc_call(target_word.astype(jnp.int32),
                  context_word.reshape(B * K).astype(jnp.int32),
                  target_table, context_table)
    return out.reshape(B, K)


# speedup vs baseline: 1.3153x; 1.3153x over previous
import functools

import jax
import jax.numpy as jnp
from jax import lax
from jax.experimental import pallas as pl
from jax.experimental.pallas import tpu as pltpu
from jax.experimental.pallas import tpu_sc as plsc

NC = 2
NS = 16
NW = NC * NS
LANES = 16

D = 64
DC = D // LANES
K = 20
CHUNK = 16
SBB = 4
N_SB = CHUNK // SBB
C_ROWS = CHUNK * K
C_STREAM = SBB * K

HALF = 1 << 19
T_BLK = 2048
N_TBLK = HALF // T_BLK


def _transpose_body(i0_ref, i1_ref, o_ref):
    o_ref[:, 0:D] = i0_ref[...].T
    o_ref[:, D:2 * D] = i1_ref[...].T


def _pack_table(table_t, V):
    last1 = (V - HALF - 1) // T_BLK
    return pl.pallas_call(
        _transpose_body,
        grid=(N_TBLK,),
        in_specs=[
            pl.BlockSpec((D, T_BLK), lambda i: (0, i)),
            pl.BlockSpec((D, T_BLK),
                         lambda i: (0, N_TBLK + jnp.minimum(i, last1))),
        ],
        out_specs=pl.BlockSpec((T_BLK, 2 * D), lambda i: (i, 0)),
        out_shape=jax.ShapeDtypeStruct((HALF, 2 * D), jnp.float32),
    )(table_t, table_t)


def _make_sc_call(B, half_v):
    b_per_w = B // NW
    n_chunks = b_per_w // CHUNK
    mesh = plsc.VectorSubcoreMesh(core_axis_name="c", subcore_axis_name="s")

    @functools.partial(
        pl.kernel,
        out_type=jax.ShapeDtypeStruct((B * K,), jnp.float32),
        mesh=mesh,
        scratch_types=[
            pltpu.VMEM((CHUNK,), jnp.int32),
            pltpu.VMEM((CHUNK,), jnp.float32),
            pltpu.VMEM((C_ROWS,), jnp.int32),
            pltpu.VMEM((C_ROWS,), jnp.float32),
            pltpu.VMEM((CHUNK, 2 * D), jnp.float32),
            pltpu.VMEM((C_ROWS, 2 * D), jnp.float32),
            pltpu.VMEM((CHUNK, D), jnp.float32),
            pltpu.VMEM((C_ROWS,), jnp.float32),
            pltpu.SemaphoreType.DMA,
        ],
        compiler_params=pltpu.CompilerParams(needs_layout_passes=False),
    )
    def sc_kernel(twm_hbm, th_hbm, cwm_hbm, ch_hbm, tpack_hbm, cpack_hbm,
                  out_hbm, tidx_v, th_v, cidx_v, ch_v, trows_v, crows_v,
                  tsel_v, outv, sem):
        wid = lax.axis_index("s") * NC + lax.axis_index("c")
        lane = lax.iota(jnp.int32, LANES)

        def chunk_body(ci, _):
            b0 = wid * b_per_w + ci * CHUNK
            pltpu.sync_copy(twm_hbm.at[pl.ds(b0, CHUNK)], tidx_v)
            pltpu.sync_copy(th_hbm.at[pl.ds(b0, CHUNK)], th_v)
            pltpu.sync_copy(cwm_hbm.at[pl.ds(b0 * K, C_ROWS)], cidx_v)
            pltpu.sync_copy(ch_hbm.at[pl.ds(b0 * K, C_ROWS)], ch_v)

            cps = [pltpu.async_copy(tpack_hbm.at[tidx_v], trows_v, sem)]
            for s in range(C_ROWS // C_STREAM):
                cps.append(pltpu.async_copy(
                    cpack_hbm.at[cidx_v.at[pl.ds(s * C_STREAM, C_STREAM)]],
                    crows_v.at[pl.ds(s * C_STREAM, C_STREAM)],
                    sem))
            for cp in cps:
                cp.wait()

            thv = th_v[...]
            for r in range(CHUNK):
                th_r = thv[r]
                for dc in range(DC):
                    lo = trows_v[r, pl.ds(dc * LANES, LANES)]
                    hi = trows_v[r, pl.ds(D + dc * LANES, LANES)]
                    tsel_v[r, pl.ds(dc * LANES, LANES)] = lo + th_r * (hi - lo)

            def sb_body(sb, _):
                tv = {}
                for bb in range(SBB):
                    for dc in range(DC):
                        tv[(bb, dc)] = tsel_v[sb * SBB + bb,
                                              pl.ds(dc * LANES, LANES)]
                for g in range(SBB * K // LANES):
                    chv = ch_v[pl.ds(sb * C_STREAM + g * LANES, LANES)]
                    outvec = jnp.zeros((LANES,), jnp.float32)
                    for j in range(LANES):
                        flat = g * LANES + j
                        bb = flat // K
                        row = sb * C_STREAM + flat
                        ch_j = chv[j]
                        acc = None
                        for dc in range(DC):
                            clo = crows_v[row, pl.ds(dc * LANES, LANES)]
                            chi = crows_v[row, pl.ds(D + dc * LANES, LANES)]
                            csel = clo + ch_j * (chi - clo)
                            term = tv[(bb, dc)] * csel
                            acc = term if acc is None else acc + term
                        outvec = jnp.where(lane == j, jnp.sum(acc), outvec)
                    outv[pl.ds(sb * C_STREAM + g * LANES, LANES)] = outvec
                return 0

            lax.fori_loop(0, N_SB, sb_body, 0, unroll=False)
            pltpu.sync_copy(outv, out_hbm.at[pl.ds(b0 * K, C_ROWS)])
            return 0

        lax.fori_loop(0, n_chunks, chunk_body, 0, unroll=False)

    return sc_kernel


def kernel(target_word, context_word, target_table, context_table):
    B, k = context_word.shape
    V = target_table.shape[0]
    assert k == K and target_table.shape[1] == D

    tpack = _pack_table(target_table.T, V)
    cpack = _pack_table(context_table.T, V)

    tw = target_word.astype(jnp.int32)
    cw = context_word.reshape(B * K).astype(jnp.int32)
    twm = tw & (HALF - 1)
    th = (tw >= HALF).astype(jnp.float32)
    cwm = cw & (HALF - 1)
    ch = (cw >= HALF).astype(jnp.float32)

    sc_call = _make_sc_call(B, HALF)
    out = sc_call(twm, th, cwm, ch, tpack, cpack)
    return out.reshape(B, K)


# speedup vs baseline: 1.4836x; 1.1280x over previous
import functools

import jax
import jax.numpy as jnp
from jax import lax
from jax.experimental import pallas as pl
from jax.experimental.pallas import tpu as pltpu
from jax.experimental.pallas import tpu_sc as plsc

NC = 2
NS = 16
NW = NC * NS
LANES = 16

D = 64
DC = D // LANES
K = 20
CHUNK = 16
SBB = 4
N_SB = CHUNK // SBB
C_ROWS = CHUNK * K
C_STREAM = SBB * K

HALF = 1 << 19
T_BLK = 2048
N_TBLK = HALF // T_BLK


def _transpose_body(i0_ref, i1_ref, o_ref):
    o_ref[:, 0:D] = i0_ref[...].T
    o_ref[:, D:2 * D] = i1_ref[...].T


def _pack_table(table_t, V):
    last1 = (V - HALF - 1) // T_BLK
    return pl.pallas_call(
        _transpose_body,
        grid=(N_TBLK,),
        in_specs=[
            pl.BlockSpec((D, T_BLK), lambda i: (0, i)),
            pl.BlockSpec((D, T_BLK),
                         lambda i: (0, N_TBLK + jnp.minimum(i, last1))),
        ],
        out_specs=pl.BlockSpec((T_BLK, 2 * D), lambda i: (i, 0)),
        out_shape=jax.ShapeDtypeStruct((HALF, 2 * D), jnp.float32),
    )(table_t, table_t)


def _make_sc_call(B):
    b_per_w = B // NW
    n_chunks = b_per_w // CHUNK
    mesh = plsc.VectorSubcoreMesh(core_axis_name="c", subcore_axis_name="s")

    @functools.partial(
        pl.kernel,
        out_type=jax.ShapeDtypeStruct((B * K,), jnp.float32),
        mesh=mesh,
        scratch_types=[
            pltpu.VMEM((CHUNK,), jnp.int32),
            pltpu.VMEM((C_ROWS,), jnp.int32),
            pltpu.VMEM((CHUNK, D), jnp.float32),
            pltpu.VMEM((C_ROWS, D), jnp.float32),
            pltpu.VMEM((C_ROWS,), jnp.float32),
            pltpu.SemaphoreType.DMA,
        ],
        compiler_params=pltpu.CompilerParams(needs_layout_passes=False,
                                             use_tc_tiling_on_sc=False),
    )
    def sc_kernel(twm_hbm, cwm_hbm, ttab_hbm, ctab_hbm, out_hbm,
                  tidx_v, cidx_v, trows_v, crows_v, outv, sem):
        wid = lax.axis_index("s") * NC + lax.axis_index("c")
        lane = lax.iota(jnp.int32, LANES)

        def chunk_body(ci, _):
            b0 = wid * b_per_w + ci * CHUNK
            pltpu.sync_copy(twm_hbm.at[pl.ds(b0, CHUNK)], tidx_v)
            pltpu.sync_copy(cwm_hbm.at[pl.ds(b0 * K, C_ROWS)], cidx_v)

            cps = [pltpu.async_copy(ttab_hbm.at[tidx_v], trows_v, sem)]
            for s in range(C_ROWS // C_STREAM):
                cps.append(pltpu.async_copy(
                    ctab_hbm.at[cidx_v.at[pl.ds(s * C_STREAM, C_STREAM)]],
                    crows_v.at[pl.ds(s * C_STREAM, C_STREAM)],
                    sem))
            for cp in cps:
                cp.wait()

            def sb_body(sb, _):
                tv = {}
                for bb in range(SBB):
                    for dc in range(DC):
                        tv[(bb, dc)] = trows_v[sb * SBB + bb,
                                               pl.ds(dc * LANES, LANES)]
                for g in range(SBB * K // LANES):
                    outvec = jnp.zeros((LANES,), jnp.float32)
                    for j in range(LANES):
                        flat = g * LANES + j
                        bb = flat // K
                        row = sb * C_STREAM + flat
                        acc = tv[(bb, 0)] * crows_v[row, pl.ds(0, LANES)]
                        for dc in range(1, DC):
                            acc = acc + tv[(bb, dc)] * crows_v[
                                row, pl.ds(dc * LANES, LANES)]
                        outvec = jnp.where(lane == j, jnp.sum(acc), outvec)
                    outv[pl.ds(sb * C_STREAM + g * LANES, LANES)] = outvec
                return 0

            lax.fori_loop(0, N_SB, sb_body, 0, unroll=False)
            pltpu.sync_copy(outv, out_hbm.at[pl.ds(b0 * K, C_ROWS)])
            return 0

        lax.fori_loop(0, n_chunks, chunk_body, 0, unroll=False)

    return sc_kernel


def kernel(target_word, context_word, target_table, context_table):
    B, k = context_word.shape
    V = target_table.shape[0]
    assert k == K and target_table.shape[1] == D

    tpack = _pack_table(target_table.T, V).reshape(2 * HALF, D)
    cpack = _pack_table(context_table.T, V).reshape(2 * HALF, D)

    tw = target_word.astype(jnp.int32)
    cw = context_word.reshape(B * K).astype(jnp.int32)
    twm = ((tw & (HALF - 1)) << 1) | (tw >> 19)
    cwm = ((cw & (HALF - 1)) << 1) | (cw >> 19)

    sc_call = _make_sc_call(B)
    out = sc_call(twm, cwm, tpack, cpack)
    return out.reshape(B, K)


# speedup vs baseline: 1.9788x; 1.3337x over previous
import functools

import jax
import jax.numpy as jnp
from jax import lax
from jax.experimental import pallas as pl
from jax.experimental.pallas import tpu as pltpu
from jax.experimental.pallas import tpu_sc as plsc

NC = 2
NS = 16
NW = NC * NS
LANES = 16

D = 64
DC = D // LANES
K = 20
CHUNK = 32
SBB = 4
N_SB = CHUNK // SBB
C_ROWS = CHUNK * K
C_STREAM = SBB * K
N_CSTREAM = C_ROWS // C_STREAM

HALF = 1 << 19
T_BLK = 4096
N_TBLK = HALF // T_BLK


def _transpose_body(i0_ref, i1_ref, o_ref):
    o_ref[:, 0:D] = i0_ref[...].T
    o_ref[:, D:2 * D] = i1_ref[...].T


def _pack_table(table_t, V):
    last1 = (V - HALF - 1) // T_BLK
    return pl.pallas_call(
        _transpose_body,
        grid=(N_TBLK,),
        in_specs=[
            pl.BlockSpec((D, T_BLK), lambda i: (0, i)),
            pl.BlockSpec((D, T_BLK),
                         lambda i: (0, N_TBLK + jnp.minimum(i, last1))),
        ],
        out_specs=pl.BlockSpec((T_BLK, 2 * D), lambda i: (i, 0)),
        out_shape=jax.ShapeDtypeStruct((HALF, 2 * D), jnp.float32),
    )(table_t, table_t)


def _make_sc_call(B):
    b_per_w = B // NW
    n_chunks = b_per_w // CHUNK
    mesh = plsc.VectorSubcoreMesh(core_axis_name="c", subcore_axis_name="s")

    @functools.partial(
        pl.kernel,
        out_type=jax.ShapeDtypeStruct((B * K,), jnp.float32),
        mesh=mesh,
        scratch_types=[
            [pltpu.VMEM((CHUNK,), jnp.int32) for _ in range(2)],
            [pltpu.VMEM((C_ROWS,), jnp.int32) for _ in range(2)],
            [pltpu.VMEM((CHUNK, D), jnp.float32) for _ in range(2)],
            [pltpu.VMEM((C_ROWS, D), jnp.float32) for _ in range(2)],
            pltpu.VMEM((C_ROWS,), jnp.float32),
            [pltpu.SemaphoreType.DMA for _ in range(2)],
            [pltpu.SemaphoreType.DMA for _ in range(2)],
        ],
        compiler_params=pltpu.CompilerParams(needs_layout_passes=False,
                                             use_tc_tiling_on_sc=False),
    )
    def sc_kernel(twm_hbm, cwm_hbm, ttab_hbm, ctab_hbm, out_hbm,
                  tidx_v, cidx_v, trows_v, crows_v, outv, semi, semg):
        wid = lax.axis_index("s") * NC + lax.axis_index("c")
        lane = lax.iota(jnp.int32, LANES)
        wbase = wid * b_per_w

        def fire_idx(s, ci):
            b0 = wbase + ci * CHUNK
            pltpu.async_copy(twm_hbm.at[pl.ds(b0, CHUNK)], tidx_v[s], semi[s])
            pltpu.async_copy(cwm_hbm.at[pl.ds(b0 * K, C_ROWS)], cidx_v[s],
                             semi[s])

        def wait_idx(s):
            pltpu.make_async_copy(twm_hbm.at[pl.ds(0, CHUNK)], tidx_v[s],
                                  semi[s]).wait()
            pltpu.make_async_copy(cwm_hbm.at[pl.ds(0, C_ROWS)], cidx_v[s],
                                  semi[s]).wait()

        def fire_gather(s):
            pltpu.async_copy(ttab_hbm.at[tidx_v[s]], trows_v[s], semg[s])
            for st in range(N_CSTREAM):
                pltpu.async_copy(
                    ctab_hbm.at[cidx_v[s].at[pl.ds(st * C_STREAM, C_STREAM)]],
                    crows_v[s].at[pl.ds(st * C_STREAM, C_STREAM)],
                    semg[s])

        def wait_gather(s):
            pltpu.make_async_copy(ttab_hbm.at[pl.ds(0, CHUNK)], trows_v[s],
                                  semg[s]).wait()
            for st in range(N_CSTREAM):
                pltpu.make_async_copy(
                    ctab_hbm.at[pl.ds(0, C_STREAM)],
                    crows_v[s].at[pl.ds(st * C_STREAM, C_STREAM)],
                    semg[s]).wait()

        def compute(s, ci):
            b0 = wbase + ci * CHUNK

            def sb_body(sb, _):
                tv = {}
                for bb in range(SBB):
                    for dc in range(DC):
                        tv[(bb, dc)] = trows_v[s][sb * SBB + bb,
                                                  pl.ds(dc * LANES, LANES)]
                for g in range(SBB * K // LANES):
                    outvec = jnp.zeros((LANES,), jnp.float32)
                    for j in range(LANES):
                        flat = g * LANES + j
                        bb = flat // K
                        row = sb * C_STREAM + flat
                        acc = tv[(bb, 0)] * crows_v[s][row, pl.ds(0, LANES)]
                        for dc in range(1, DC):
                            acc = acc + tv[(bb, dc)] * crows_v[s][
                                row, pl.ds(dc * LANES, LANES)]
                        outvec = jnp.where(lane == j, jnp.sum(acc), outvec)
                    outv[pl.ds(sb * C_STREAM + g * LANES, LANES)] = outvec
                return 0

            lax.fori_loop(0, N_SB, sb_body, 0, unroll=False)
            pltpu.sync_copy(outv, out_hbm.at[pl.ds(b0 * K, C_ROWS)])

        last = n_chunks - 1
        fire_idx(0, 0)
        wait_idx(0)
        fire_gather(0)
        fire_idx(1, 1)

        def pair_body(p, _):
            c0 = p * 2

            def step(s, c):
                wait_gather(s)
                fire_idx(s, jnp.minimum(c + 2, last))
                wait_idx(1 - s)
                fire_gather(1 - s)
                compute(s, c)

            step(0, c0)
            step(1, c0 + 1)
            return 0

        lax.fori_loop(0, n_chunks // 2, pair_body, 0, unroll=False)
        wait_gather(0)
        wait_idx(1)

    return sc_kernel


def kernel(target_word, context_word, target_table, context_table):
    B, k = context_word.shape
    V = target_table.shape[0]
    assert k == K and target_table.shape[1] == D

    tpack = _pack_table(target_table.T, V).reshape(2 * HALF, D)
    cpack = _pack_table(context_table.T, V).reshape(2 * HALF, D)

    tw = target_word.astype(jnp.int32)
    cw = context_word.reshape(B * K).astype(jnp.int32)
    twm = ((tw & (HALF - 1)) << 1) | (tw >> 19)
    cwm = ((cw & (HALF - 1)) << 1) | (cw >> 19)

    sc_call = _make_sc_call(B)
    out = sc_call(twm, cwm, tpack, cpack)
    return out.reshape(B, K)


# speedup vs baseline: 2.0105x; 1.0160x over previous
import functools

import jax
import jax.numpy as jnp
from jax import lax
from jax.experimental import pallas as pl
from jax.experimental.pallas import tpu as pltpu
from jax.experimental.pallas import tpu_sc as plsc

NC = 2
NS = 16
NW = NC * NS
LANES = 16

D = 64
DW = D // 2
K = 20
CHUNK = 32
SBB = 4
N_SB = CHUNK // SBB
C_ROWS = CHUNK * K
C_STREAM = SBB * K
N_CSTREAM = C_ROWS // C_STREAM

VPAD = 1 << 20
T_BLK = 8192
N_TBLK = VPAD // T_BLK


def _pack_body(lo_ref, hi_ref, o_ref):
    lob = lo_ref[...].astype(jnp.bfloat16)
    hib = hi_ref[...].astype(jnp.bfloat16)
    lo = lax.bitcast_convert_type(lob, jnp.uint16).astype(jnp.uint32)
    hi = lax.bitcast_convert_type(hib, jnp.uint16).astype(jnp.uint32)
    z = (lo | (hi << 16)).astype(jnp.int32)
    q_blk = T_BLK // 4
    for q in range(4):
        zq = z[:, q * q_blk:(q + 1) * q_blk]
        o_ref[:, DW * q:DW * (q + 1)] = zq.T


def _pack_table(table_t, V):
    last = (V - 1) // T_BLK
    return pl.pallas_call(
        _pack_body,
        grid=(N_TBLK,),
        in_specs=[
            pl.BlockSpec((DW, T_BLK), lambda i: (0, jnp.minimum(i, last))),
            pl.BlockSpec((DW, T_BLK), lambda i: (1, jnp.minimum(i, last))),
        ],
        out_specs=pl.BlockSpec((T_BLK // 4, 4 * DW), lambda i: (i, 0)),
        out_shape=jax.ShapeDtypeStruct((VPAD // 4, 4 * DW), jnp.int32),
    )(table_t, table_t)


def _make_sc_call(B):
    b_per_w = B // NW
    n_chunks = b_per_w // CHUNK
    mesh = plsc.VectorSubcoreMesh(core_axis_name="c", subcore_axis_name="s")

    @functools.partial(
        pl.kernel,
        out_type=jax.ShapeDtypeStruct((B * K,), jnp.float32),
        mesh=mesh,
        scratch_types=[
            [pltpu.VMEM((CHUNK,), jnp.int32) for _ in range(2)],
            [pltpu.VMEM((C_ROWS,), jnp.int32) for _ in range(2)],
            [pltpu.VMEM((CHUNK, DW), jnp.int32) for _ in range(2)],
            [pltpu.VMEM((C_ROWS, DW), jnp.int32) for _ in range(2)],
            pltpu.VMEM((C_ROWS,), jnp.float32),
            [pltpu.SemaphoreType.DMA for _ in range(2)],
            [pltpu.SemaphoreType.DMA for _ in range(2)],
        ],
        compiler_params=pltpu.CompilerParams(needs_layout_passes=False,
                                             use_tc_tiling_on_sc=False),
    )
    def sc_kernel(twm_hbm, cwm_hbm, ttab_hbm, ctab_hbm, out_hbm,
                  tidx_v, cidx_v, trows_v, crows_v, outv, semi, semg):
        wid = lax.axis_index("s") * NC + lax.axis_index("c")
        lane = lax.iota(jnp.int32, LANES)
        wbase = wid * b_per_w

        def unpack2(words):
            bf = plsc.bitcast(words, jnp.bfloat16)
            return plsc.unpack(bf, format=plsc.PackFormat.INTERLEAVED)

        def fire_idx(s, ci):
            b0 = wbase + ci * CHUNK
            pltpu.async_copy(twm_hbm.at[pl.ds(b0, CHUNK)], tidx_v[s], semi[s])
            pltpu.async_copy(cwm_hbm.at[pl.ds(b0 * K, C_ROWS)], cidx_v[s],
                             semi[s])

        def wait_idx(s):
            pltpu.make_async_copy(twm_hbm.at[pl.ds(0, CHUNK)], tidx_v[s],
                                  semi[s]).wait()
            pltpu.make_async_copy(cwm_hbm.at[pl.ds(0, C_ROWS)], cidx_v[s],
                                  semi[s]).wait()

        def fire_gather(s):
            pltpu.async_copy(ttab_hbm.at[tidx_v[s]], trows_v[s], semg[s])
            for st in range(N_CSTREAM):
                pltpu.async_copy(
                    ctab_hbm.at[cidx_v[s].at[pl.ds(st * C_STREAM, C_STREAM)]],
                    crows_v[s].at[pl.ds(st * C_STREAM, C_STREAM)],
                    semg[s])

        def wait_gather(s):
            pltpu.make_async_copy(ttab_hbm.at[pl.ds(0, CHUNK)], trows_v[s],
                                  semg[s]).wait()
            for st in range(N_CSTREAM):
                pltpu.make_async_copy(
                    ctab_hbm.at[pl.ds(0, C_STREAM)],
                    crows_v[s].at[pl.ds(st * C_STREAM, C_STREAM)],
                    semg[s]).wait()

        def compute(s, ci):
            b0 = wbase + ci * CHUNK

            def sb_body(sb, _):
                tv = {}
                for bb in range(SBB):
                    for w in range(2):
                        words = trows_v[s][sb * SBB + bb,
                                           pl.ds(w * LANES, LANES)]
                        tv[(bb, 2 * w)], tv[(bb, 2 * w + 1)] = unpack2(words)
                for g in range(SBB * K // LANES):
                    outvec = jnp.zeros((LANES,), jnp.float32)
                    for j in range(LANES):
                        flat = g * LANES + j
                        bb = flat // K
                        row = sb * C_STREAM + flat
                        acc = None
                        for w in range(2):
                            words = crows_v[s][row, pl.ds(w * LANES, LANES)]
                            ca, cb = unpack2(words)
                            term = tv[(bb, 2 * w)] * ca + tv[(bb, 2 * w + 1)] * cb
                            acc = term if acc is None else acc + term
                        outvec = jnp.where(lane == j, jnp.sum(acc), outvec)
                    outv[pl.ds(sb * C_STREAM + g * LANES, LANES)] = outvec
                return 0

            lax.fori_loop(0, N_SB, sb_body, 0, unroll=False)
            pltpu.sync_copy(outv, out_hbm.at[pl.ds(b0 * K, C_ROWS)])

        last = n_chunks - 1
        fire_idx(0, 0)
        wait_idx(0)
        fire_gather(0)
        fire_idx(1, 1)

        def pair_body(p, _):
            c0 = p * 2

            def step(s, c):
                wait_gather(s)
                fire_idx(s, jnp.minimum(c + 2, last))
                wait_idx(1 - s)
                fire_gather(1 - s)
                compute(s, c)

            step(0, c0)
            step(1, c0 + 1)
            return 0

        lax.fori_loop(0, n_chunks // 2, pair_body, 0, unroll=False)
        wait_gather(0)
        wait_idx(1)

    return sc_kernel


def kernel(target_word, context_word, target_table, context_table):
    B, k = context_word.shape
    V = target_table.shape[0]
    assert k == K and target_table.shape[1] == D

    tpack = _pack_table(target_table.T, V).reshape(VPAD, DW)
    cpack = _pack_table(context_table.T, V).reshape(VPAD, DW)

    def linrow(v):
        return (v & ~8191) | ((v & 2047) << 2) | ((v >> 11) & 3)

    twm = linrow(target_word.astype(jnp.int32))
    cwm = linrow(context_word.reshape(B * K).astype(jnp.int32))

    sc_call = _make_sc_call(B)
    out = sc_call(twm, cwm, tpack, cpack)
    return out.reshape(B, K)


# speedup vs baseline: 2.0642x; 1.0267x over previous
import functools

import jax
import jax.numpy as jnp
from jax import lax
from jax.experimental import pallas as pl
from jax.experimental.pallas import tpu as pltpu
from jax.experimental.pallas import tpu_sc as plsc

NC = 2
NS = 16
NW = NC * NS
LANES = 16

D = 64
DW = D // 2
K = 20
CHUNK = 32
SBB = 4
N_SB = CHUNK // SBB
C_ROWS = CHUNK * K
C_STREAM = SBB * K
N_CSTREAM = C_ROWS // C_STREAM

VPAD = 1 << 20
T_BLK = 16384
N_TBLK = VPAD // T_BLK


def _pack_body(lo_ref, hi_ref, o_ref):
    q_blk = T_BLK // 4
    eye = jnp.eye(DW, dtype=jnp.bfloat16)
    dn = (((0,), (0,)), ((), ()))
    for q in range(4):
        cols = pl.ds(q * q_blk, q_blk)
        lob = lo_ref[:, cols].astype(jnp.bfloat16)
        hib = hi_ref[:, cols].astype(jnp.bfloat16)
        lot = lax.dot_general(
            lob, eye, dn,
            preferred_element_type=jnp.float32).astype(jnp.bfloat16)
        hit = lax.dot_general(
            hib, eye, dn,
            preferred_element_type=jnp.float32).astype(jnp.bfloat16)
        lo = lax.bitcast_convert_type(lot, jnp.uint16).astype(jnp.uint32)
        hi = lax.bitcast_convert_type(hit, jnp.uint16).astype(jnp.uint32)
        o_ref[:, DW * q:DW * (q + 1)] = (lo | (hi << 16)).astype(jnp.int32)


def _pack_table(table_t, V):
    last = (V - 1) // T_BLK
    return pl.pallas_call(
        _pack_body,
        grid=(N_TBLK,),
        in_specs=[
            pl.BlockSpec((DW, T_BLK), lambda i: (0, jnp.minimum(i, last))),
            pl.BlockSpec((DW, T_BLK), lambda i: (1, jnp.minimum(i, last))),
        ],
        out_specs=pl.BlockSpec((T_BLK // 4, 4 * DW), lambda i: (i, 0)),
        out_shape=jax.ShapeDtypeStruct((VPAD // 4, 4 * DW), jnp.int32),
    )(table_t, table_t)


def _make_sc_call(B):
    b_per_w = B // NW
    n_chunks = b_per_w // CHUNK
    mesh = plsc.VectorSubcoreMesh(core_axis_name="c", subcore_axis_name="s")

    @functools.partial(
        pl.kernel,
        out_type=jax.ShapeDtypeStruct((B * K,), jnp.float32),
        mesh=mesh,
        scratch_types=[
            [pltpu.VMEM((CHUNK,), jnp.int32) for _ in range(2)],
            [pltpu.VMEM((C_ROWS,), jnp.int32) for _ in range(2)],
            [pltpu.VMEM((CHUNK, DW), jnp.int32) for _ in range(2)],
            [pltpu.VMEM((C_ROWS, DW), jnp.int32) for _ in range(2)],
            pltpu.VMEM((C_ROWS,), jnp.float32),
            [pltpu.SemaphoreType.DMA for _ in range(2)],
            [pltpu.SemaphoreType.DMA for _ in range(2)],
        ],
        compiler_params=pltpu.CompilerParams(needs_layout_passes=False,
                                             use_tc_tiling_on_sc=False),
    )
    def sc_kernel(twm_hbm, cwm_hbm, ttab_hbm, ctab_hbm, out_hbm,
                  tidx_v, cidx_v, trows_v, crows_v, outv, semi, semg):
        wid = lax.axis_index("s") * NC + lax.axis_index("c")
        lane = lax.iota(jnp.int32, LANES)
        wbase = wid * b_per_w

        def unpack2(words):
            bf = plsc.bitcast(words, jnp.bfloat16)
            return plsc.unpack(bf, format=plsc.PackFormat.INTERLEAVED)

        def fire_idx(s, ci):
            b0 = wbase + ci * CHUNK
            pltpu.async_copy(twm_hbm.at[pl.ds(b0, CHUNK)], tidx_v[s], semi[s])
            pltpu.async_copy(cwm_hbm.at[pl.ds(b0 * K, C_ROWS)], cidx_v[s],
                             semi[s])

        def wait_idx(s):
            pltpu.make_async_copy(twm_hbm.at[pl.ds(0, CHUNK)], tidx_v[s],
                                  semi[s]).wait()
            pltpu.make_async_copy(cwm_hbm.at[pl.ds(0, C_ROWS)], cidx_v[s],
                                  semi[s]).wait()

        def fire_gather(s):
            pltpu.async_copy(ttab_hbm.at[tidx_v[s]], trows_v[s], semg[s])
            for st in range(N_CSTREAM):
                pltpu.async_copy(
                    ctab_hbm.at[cidx_v[s].at[pl.ds(st * C_STREAM, C_STREAM)]],
                    crows_v[s].at[pl.ds(st * C_STREAM, C_STREAM)],
                    semg[s])

        def wait_gather(s):
            pltpu.make_async_copy(ttab_hbm.at[pl.ds(0, CHUNK)], trows_v[s],
                                  semg[s]).wait()
            for st in range(N_CSTREAM):
                pltpu.make_async_copy(
                    ctab_hbm.at[pl.ds(0, C_STREAM)],
                    crows_v[s].at[pl.ds(st * C_STREAM, C_STREAM)],
                    semg[s]).wait()

        def compute(s, ci):
            b0 = wbase + ci * CHUNK

            def sb_body(sb, _):
                tv = {}
                for bb in range(SBB):
                    for w in range(2):
                        words = trows_v[s][sb * SBB + bb,
                                           pl.ds(w * LANES, LANES)]
                        tv[(bb, 2 * w)], tv[(bb, 2 * w + 1)] = unpack2(words)
                for g in range(SBB * K // LANES):
                    outvec = jnp.zeros((LANES,), jnp.float32)
                    for j in range(LANES):
                        flat = g * LANES + j
                        bb = flat // K
                        row = sb * C_STREAM + flat
                        acc = None
                        for w in range(2):
                            words = crows_v[s][row, pl.ds(w * LANES, LANES)]
                            ca, cb = unpack2(words)
                            term = tv[(bb, 2 * w)] * ca + tv[(bb, 2 * w + 1)] * cb
                            acc = term if acc is None else acc + term
                        outvec = jnp.where(lane == j, jnp.sum(acc), outvec)
                    outv[pl.ds(sb * C_STREAM + g * LANES, LANES)] = outvec
                return 0

            lax.fori_loop(0, N_SB, sb_body, 0, unroll=False)
            pltpu.sync_copy(outv, out_hbm.at[pl.ds(b0 * K, C_ROWS)])

        last = n_chunks - 1
        fire_idx(0, 0)
        wait_idx(0)
        fire_gather(0)
        fire_idx(1, 1)

        def pair_body(p, _):
            c0 = p * 2

            def step(s, c):
                wait_gather(s)
                fire_idx(s, jnp.minimum(c + 2, last))
                wait_idx(1 - s)
                fire_gather(1 - s)
                compute(s, c)

            step(0, c0)
            step(1, c0 + 1)
            return 0

        lax.fori_loop(0, n_chunks // 2, pair_body, 0, unroll=False)
        wait_gather(0)
        wait_idx(1)

    return sc_kernel


def kernel(target_word, context_word, target_table, context_table):
    B, k = context_word.shape
    V = target_table.shape[0]
    assert k == K and target_table.shape[1] == D

    tpack = _pack_table(target_table.T, V).reshape(VPAD, DW)
    cpack = _pack_table(context_table.T, V).reshape(VPAD, DW)

    q_blk = T_BLK // 4
    q_shift = q_blk.bit_length() - 1

    def linrow(v):
        return (v & ~(T_BLK - 1)) | ((v & (q_blk - 1)) << 2) | (
            (v >> q_shift) & 3)

    twm = linrow(target_word.astype(jnp.int32))
    cwm = linrow(context_word.reshape(B * K).astype(jnp.int32))

    sc_call = _make_sc_call(B)
    out = sc_call(twm, cwm, tpack, cpack)
    return out.reshape(B, K)
